# baseline (speedup 1.0000x reference)
import functools

import jax
import jax.numpy as jnp
from jax import lax
from jax.experimental import pallas as pl
from jax.experimental.pallas import tpu as pltpu
from jax.experimental.pallas import tpu_sc as plsc

N = 10000
E = 320000
D = 128
NPAD = 10240
AW = 136
KP = 32
NB = 4
SCH = 16
NSUP = 20
EPT = E // 32
CH = NSUP * SCH
EPTP = CH * KP
RPT = NPAD // 16


def _prep(X2, W1, W2, U, V):
    def body(x_ref, w1_ref, w2_ref, u_ref, v_ref, x_out, xv2t_out, xu_out):
        x = jnp.dot(x_ref[...], w1_ref[...], preferred_element_type=jnp.float32)
        x_out[...] = x
        xv2t_out[...] = jnp.dot(
            x, v_ref[...], preferred_element_type=jnp.float32) * w2_ref[...]
        xu_out[...] = jnp.dot(x, u_ref[...], preferred_element_type=jnp.float32)

    B = 2000
    return pl.pallas_call(
        body,
        grid=(N // B,),
        in_specs=[
            pl.BlockSpec((B, D), lambda i: (i, 0)),
            pl.BlockSpec((D, D), lambda i: (0, 0)),
            pl.BlockSpec((1, D), lambda i: (0, 0)),
            pl.BlockSpec((D, D), lambda i: (0, 0)),
            pl.BlockSpec((D, D), lambda i: (0, 0)),
        ],
        out_specs=[pl.BlockSpec((B, D), lambda i: (i, 0))] * 3,
        out_shape=[jax.ShapeDtypeStruct((N, D), jnp.float32)] * 3,
    )(X2, W1, W2, U, V)


def _sc_scatter(xv2t, src3, dst3, wp):
    mesh = plsc.VectorSubcoreMesh(core_axis_name="c", subcore_axis_name="s")

    @functools.partial(
        pl.kernel,
        out_type=jax.ShapeDtypeStruct((2, NPAD, AW), jnp.float32),
        mesh=mesh,
        compiler_params=pltpu.CompilerParams(
            needs_layout_passes=False, use_tc_tiling_on_sc=False),
        scratch_types=[
            pltpu.VMEM((2, SCH, KP), jnp.int32),
            pltpu.VMEM((2, SCH, KP), jnp.int32),
            pltpu.VMEM((2, SCH * KP), jnp.float32),
            pltpu.VMEM((NB, KP, D), jnp.float32),
            pltpu.VMEM((NB, KP, AW), jnp.float32),
            pltpu.VMEM_SHARED((NPAD, AW), jnp.float32),
            pltpu.SemaphoreType.DMA,
            pltpu.SemaphoreType.DMA,
            pltpu.SemaphoreType.DMA,
            pltpu.SemaphoreType.DMA,
            pltpu.SemaphoreType.DMA,
            pltpu.SemaphoreType.DMA,
            pltpu.SemaphoreType.DMA,
            pltpu.SemaphoreType.DMA,
            pltpu.SemaphoreType.DMA,
            pltpu.SemaphoreType.DMA,
        ],
    )
    def k(xv2t_hbm, src_hbm, dst_hbm, w_hbm, out_hbm,
          src_sup, dst_sup, w_sup, gbuf, rows, accum,
          sg0, sg1, sg2, sg3, ss0, ss1, ss2, ss3, sm0, sm1):
        c = lax.axis_index("c")
        s = lax.axis_index("s")
        tid = c * 16 + s
        sem_g = (sg0, sg1, sg2, sg3)
        sem_s = (ss0, ss1, ss2, ss3)
        sem_m = (sm0, sm1)

        zero16 = jnp.zeros((16,), jnp.float32)

        @plsc.parallel_loop(0, KP, unroll=4)
        def _(e):
            for b in range(NB):
                for j in range(8):
                    rows[b, e, pl.ds(j * 16, 16)] = zero16
                rows[b, e, pl.ds(AW - 16, 16)] = zero16

        for kk in range(RPT // KP):
            pltpu.sync_copy(rows.at[0], accum.at[pl.ds(s * RPT + kk * KP, KP)])

        tail = jnp.where(lax.iota(jnp.int32, 16) == 8,
                         jnp.float32(1.0), jnp.float32(0.0))

        @plsc.parallel_loop(0, KP, unroll=4)
        def _(e):
            for b in range(NB):
                rows[b, e, pl.ds(AW - 16, 16)] = tail

        plsc.subcore_barrier()

        def meta_start(ss, p):
            pltpu.async_copy(src_hbm.at[tid, ss], src_sup.at[p], sem_m[p])
            pltpu.async_copy(dst_hbm.at[tid, ss], dst_sup.at[p], sem_m[p])
            pltpu.async_copy(w_hbm.at[tid, ss], w_sup.at[p], sem_m[p])

        def meta_wait(ss, p):
            pltpu.make_async_copy(
                src_hbm.at[tid, ss], src_sup.at[p], sem_m[p]).wait()
            pltpu.make_async_copy(
                dst_hbm.at[tid, ss], dst_sup.at[p], sem_m[p]).wait()
            pltpu.make_async_copy(
                w_hbm.at[tid, ss], w_sup.at[p], sem_m[p]).wait()

        def gather_start(p, j, b):
            pltpu.async_copy(xv2t_hbm.at[src_sup.at[p, j]], gbuf.at[b],
                             sem_g[b])

        def gather_wait(b):
            pltpu.make_async_copy(
                xv2t_hbm.at[src_sup.at[0, 0]], gbuf.at[b], sem_g[b]).wait()

        def scatter_start(p, j, b):
            pltpu.async_copy(rows.at[b], accum.at[dst_sup.at[p, j]],
                             sem_s[b], add=True)

        def scatter_wait(b):
            pltpu.make_async_copy(
                rows.at[b], accum.at[pl.ds(0, KP)], sem_s[b]).wait()

        def scale(p, j, b):
            p16 = jnp.full((16,), p, jnp.int32)
            jbase = jnp.full((16,), j * KP, jnp.int32)

            @plsc.parallel_loop(0, KP, unroll=8)
            def _(e):
                wspl = plsc.load_gather(w_sup, [p16, jbase + e])
                for jj in range(8):
                    rows[b, e, pl.ds(jj * 16, 16)] = (
                        gbuf[b, e, pl.ds(jj * 16, 16)] * wspl)

        meta_start(0, 0)
        meta_wait(0, 0)
        for b in range(NB):
            gather_start(0, b, b)

        def suppair(g, carry):
            for sp in range(2):
                ss = 2 * g + sp
                for j in range(SCH):
                    b = j % NB
                    if j == NB:
                        meta_start(jnp.minimum(ss + 1, NSUP - 1), sp ^ 1)
                    if j >= NB:
                        scatter_wait(b)
                    else:
                        @pl.when(ss > 0)
                        def _():
                            scatter_wait(b)
                    gather_wait(b)
                    scale(sp, j, b)
                    if j == SCH - NB:
                        meta_wait(jnp.minimum(ss + 1, NSUP - 1), sp ^ 1)
                    if j < SCH - NB:
                        gather_start(sp, j + NB, b)
                    else:
                        gather_start(sp ^ 1, j + NB - SCH, b)
                    scatter_start(sp, j, b)
            return carry

        lax.fori_loop(0, NSUP // 2, suppair, 0)

        for b in range(NB):
            scatter_wait(b)
            gather_wait(b)
        plsc.subcore_barrier()
        pltpu.sync_copy(accum.at[pl.ds(s * RPT, RPT)],
                        out_hbm.at[c, pl.ds(s * RPT, RPT)])

    return k(xv2t, src3, dst3, wp)


def _post(x, xu, acc):
    def body(x_ref, xu_ref, a_ref, o_ref):
        a = a_ref[0] + a_ref[1]
        ssum = a[:, :D]
        cnt = a[:, D:D + 1]
        denom = jnp.maximum(cnt, 1.0)
        ao = xu_ref[...] + ssum / denom
        mu = jnp.mean(ao, axis=1, keepdims=True)
        var = jnp.mean(jnp.square(ao - mu), axis=1, keepdims=True)
        normed = (ao - mu) * lax.rsqrt(var + jnp.float32(1e-5))
        o_ref[...] = x_ref[...] + jnp.maximum(normed, 0.0)

    B = 2000
    return pl.pallas_call(
        body,
        grid=(N // B,),
        in_specs=[
            pl.BlockSpec((B, D), lambda i: (i, 0)),
            pl.BlockSpec((B, D), lambda i: (i, 0)),
            pl.BlockSpec((2, B, AW), lambda i: (0, i, 0)),
        ],
        out_specs=pl.BlockSpec((B, D), lambda i: (i, 0)),
        out_shape=jax.ShapeDtypeStruct((N, D), jnp.float32),
    )(x, xu, acc)


def kernel(X, edge_index, edge_weight, W1, W2, U, V):
    X2 = X.reshape(N, D)
    x, xv2t, xu = _prep(X2, W1, W2, U, V)

    pad = EPTP - EPT
    src3 = jnp.concatenate(
        [edge_index[0].reshape(32, EPT),
         jnp.zeros((32, pad), jnp.int32)], axis=1).reshape(32, NSUP, SCH, KP)
    dst3 = jnp.concatenate(
        [edge_index[1].reshape(32, EPT),
         jnp.full((32, pad), N, jnp.int32)], axis=1).reshape(32, NSUP, SCH, KP)
    wp = jnp.concatenate(
        [edge_weight.reshape(32, EPT),
         jnp.zeros((32, pad), jnp.float32)], axis=1).reshape(32, NSUP, SCH * KP)

    acc = _sc_scatter(xv2t, src3, dst3, wp)
    out = _post(x, xu, acc)
    return out.reshape(1, N, D)


# speedup vs baseline: 1.4502x; 1.4502x over previous
import functools

import jax
import jax.numpy as jnp
from jax import lax
from jax.experimental import pallas as pl
from jax.experimental.pallas import tpu as pltpu
from jax.experimental.pallas import tpu_sc as plsc

N = 10000
E = 320000
D = 128
NPAD = 10240
AW = 136
KP = 64
SCH = 8
NSUP = 20
EPT = E // 32
CH = NSUP * SCH
EPTP = CH * KP
RPT = NPAD // 16


def _prep(X2, W1, W2, U, V):
    def body(x_ref, w1_ref, w2_ref, u_ref, v_ref, x_out, xv2t_out, xu_out):
        x = jnp.dot(x_ref[...], w1_ref[...], preferred_element_type=jnp.float32)
        x_out[...] = x
        xv2t_out[...] = jnp.dot(
            x, v_ref[...], preferred_element_type=jnp.float32) * w2_ref[...]
        xu_out[...] = jnp.dot(x, u_ref[...], preferred_element_type=jnp.float32)

    B = 2000
    return pl.pallas_call(
        body,
        grid=(N // B,),
        in_specs=[
            pl.BlockSpec((B, D), lambda i: (i, 0)),
            pl.BlockSpec((D, D), lambda i: (0, 0)),
            pl.BlockSpec((1, D), lambda i: (0, 0)),
            pl.BlockSpec((D, D), lambda i: (0, 0)),
            pl.BlockSpec((D, D), lambda i: (0, 0)),
        ],
        out_specs=[pl.BlockSpec((B, D), lambda i: (i, 0))] * 3,
        out_shape=[jax.ShapeDtypeStruct((N, D), jnp.float32)] * 3,
    )(X2, W1, W2, U, V)


def _sc_scatter(xv2t, src3, dst3, wp):
    mesh = plsc.VectorSubcoreMesh(core_axis_name="c", subcore_axis_name="s")

    @functools.partial(
        pl.kernel,
        out_type=jax.ShapeDtypeStruct((2, NPAD, AW), jnp.float32),
        mesh=mesh,
        compiler_params=pltpu.CompilerParams(
            needs_layout_passes=False, use_tc_tiling_on_sc=False),
        scratch_types=[
            pltpu.VMEM((2, SCH, KP), jnp.int32),
            pltpu.VMEM((2, SCH, KP), jnp.int32),
            pltpu.VMEM((2, SCH * KP), jnp.float32),
            pltpu.VMEM((2, KP, D), jnp.bfloat16),
            pltpu.VMEM((2, KP, AW), jnp.float32),
            pltpu.VMEM_SHARED((NPAD, AW), jnp.float32),
            pltpu.SemaphoreType.DMA,
            pltpu.SemaphoreType.DMA,
            pltpu.SemaphoreType.DMA,
            pltpu.SemaphoreType.DMA,
            pltpu.SemaphoreType.DMA,
            pltpu.SemaphoreType.DMA,
        ],
    )
    def k(xv2t_hbm, src_hbm, dst_hbm, w_hbm, out_hbm,
          src_sup, dst_sup, w_sup, gbuf, rows, accum,
          sg0, sg1, ss0, ss1, sm0, sm1):
        c = lax.axis_index("c")
        s = lax.axis_index("s")
        tid = c * 16 + s
        sem_g = (sg0, sg1)
        sem_s = (ss0, ss1)
        sem_m = (sm0, sm1)

        zero16 = jnp.zeros((16,), jnp.float32)

        @plsc.parallel_loop(0, KP, unroll=4)
        def _(e):
            for b in range(2):
                for j in range(8):
                    rows[b, e, pl.ds(j * 16, 16)] = zero16
                rows[b, e, pl.ds(AW - 16, 16)] = zero16

        for kk in range(RPT // KP):
            pltpu.sync_copy(rows.at[0], accum.at[pl.ds(s * RPT + kk * KP, KP)])

        tail = jnp.where(lax.iota(jnp.int32, 16) == 8,
                         jnp.float32(1.0), jnp.float32(0.0))

        @plsc.parallel_loop(0, KP, unroll=4)
        def _(e):
            rows[0, e, pl.ds(AW - 16, 16)] = tail
            rows[1, e, pl.ds(AW - 16, 16)] = tail

        plsc.subcore_barrier()

        def meta_start(ss, p):
            pltpu.async_copy(src_hbm.at[tid, ss], src_sup.at[p], sem_m[p])
            pltpu.async_copy(dst_hbm.at[tid, ss], dst_sup.at[p], sem_m[p])
            pltpu.async_copy(w_hbm.at[tid, ss], w_sup.at[p], sem_m[p])

        def meta_wait(ss, p):
            pltpu.make_async_copy(
                src_hbm.at[tid, ss], src_sup.at[p], sem_m[p]).wait()
            pltpu.make_async_copy(
                dst_hbm.at[tid, ss], dst_sup.at[p], sem_m[p]).wait()
            pltpu.make_async_copy(
                w_hbm.at[tid, ss], w_sup.at[p], sem_m[p]).wait()

        def gather_start(p, j, b):
            pltpu.async_copy(xv2t_hbm.at[src_sup.at[p, j]], gbuf.at[b],
                             sem_g[b])

        def gather_wait(b):
            pltpu.make_async_copy(
                xv2t_hbm.at[src_sup.at[0, 0]], gbuf.at[b], sem_g[b]).wait()

        def scatter_start(p, j, b):
            pltpu.async_copy(rows.at[b], accum.at[dst_sup.at[p, j]],
                             sem_s[b], add=True)

        def scatter_wait(b):
            pltpu.make_async_copy(
                rows.at[b], accum.at[pl.ds(0, KP)], sem_s[b]).wait()

        def scale(p, j, b):
            p16 = jnp.full((16,), p, jnp.int32)
            jbase = jnp.full((16,), j * KP, jnp.int32)

            @plsc.parallel_loop(0, KP, unroll=8)
            def _(e):
                wspl = plsc.load_gather(w_sup, [p16, jbase + e])
                for jj in range(4):
                    x32 = plsc.bitcast(gbuf[b, e, pl.ds(jj * 32, 32)],
                                       jnp.int32)
                    lo = plsc.bitcast(
                        lax.shift_left(x32, jnp.int32(16)), jnp.float32)
                    hi = plsc.bitcast(
                        lax.bitwise_and(x32, jnp.int32(-65536)), jnp.float32)
                    rows[b, e, pl.ds(jj * 32, 16)] = lo * wspl
                    rows[b, e, pl.ds(jj * 32 + 16, 16)] = hi * wspl

        meta_start(0, 0)
        meta_wait(0, 0)
        gather_start(0, 0, 0)
        gather_start(0, 1, 1)

        def suppair(g, carry):
            for sp in range(2):
                ss = 2 * g + sp
                for j in range(SCH):
                    b = j & 1
                    if j == 2:
                        meta_start(jnp.minimum(ss + 1, NSUP - 1), sp ^ 1)
                    if j >= 2:
                        scatter_wait(b)
                    else:
                        @pl.when(ss > 0)
                        def _():
                            scatter_wait(b)
                    gather_wait(b)
                    scale(sp, j, b)
                    if j == 6:
                        meta_wait(jnp.minimum(ss + 1, NSUP - 1), sp ^ 1)
                    if j < SCH - 2:
                        gather_start(sp, j + 2, b)
                    else:
                        gather_start(sp ^ 1, j + 2 - SCH, b)
                    scatter_start(sp, j, b)
            return carry

        lax.fori_loop(0, NSUP // 2, suppair, 0)

        scatter_wait(0)
        scatter_wait(1)
        gather_wait(0)
        gather_wait(1)
        plsc.subcore_barrier()
        pltpu.sync_copy(accum.at[pl.ds(s * RPT, RPT)],
                        out_hbm.at[c, pl.ds(s * RPT, RPT)])

    return k(xv2t, src3, dst3, wp)


def _post(x, xu, acc):
    def body(x_ref, xu_ref, a_ref, o_ref):
        a = a_ref[0] + a_ref[1]
        ssum = a[:, :D]
        cnt = a[:, D:D + 1]
        denom = jnp.maximum(cnt, 1.0)
        ao = xu_ref[...] + ssum / denom
        mu = jnp.mean(ao, axis=1, keepdims=True)
        var = jnp.mean(jnp.square(ao - mu), axis=1, keepdims=True)
        normed = (ao - mu) * lax.rsqrt(var + jnp.float32(1e-5))
        o_ref[...] = x_ref[...] + jnp.maximum(normed, 0.0)

    B = 2000
    return pl.pallas_call(
        body,
        grid=(N // B,),
        in_specs=[
            pl.BlockSpec((B, D), lambda i: (i, 0)),
            pl.BlockSpec((B, D), lambda i: (i, 0)),
            pl.BlockSpec((2, B, AW), lambda i: (0, i, 0)),
        ],
        out_specs=pl.BlockSpec((B, D), lambda i: (i, 0)),
        out_shape=jax.ShapeDtypeStruct((N, D), jnp.float32),
    )(x, xu, acc)


def kernel(X, edge_index, edge_weight, W1, W2, U, V):
    X2 = X.reshape(N, D)
    x, xv2t, xu = _prep(X2, W1, W2, U, V)

    pad = EPTP - EPT
    src3 = jnp.concatenate(
        [edge_index[0].reshape(32, EPT),
         jnp.zeros((32, pad), jnp.int32)], axis=1).reshape(32, NSUP, SCH, KP)
    dst3 = jnp.concatenate(
        [edge_index[1].reshape(32, EPT),
         jnp.full((32, pad), N, jnp.int32)], axis=1).reshape(32, NSUP, SCH, KP)
    wp = jnp.concatenate(
        [edge_weight.reshape(32, EPT),
         jnp.zeros((32, pad), jnp.float32)], axis=1).reshape(32, NSUP, SCH * KP)

    xv2b = (xv2t.reshape(N, 4, 2, 16).swapaxes(2, 3).reshape(N, D)
            .astype(jnp.bfloat16))
    acc = _sc_scatter(xv2b, src3, dst3, wp)
    out = _post(x, xu, acc)
    return out.reshape(1, N, D)


# speedup vs baseline: 1.4598x; 1.0067x over previous
import functools

import jax
import jax.numpy as jnp
from jax import lax
from jax.experimental import pallas as pl
from jax.experimental.pallas import tpu as pltpu
from jax.experimental.pallas import tpu_sc as plsc

N = 10000
E = 320000
D = 128
NPAD = 10240
AW = 136
KP = 80
SCH = 8
NSUP = 16
EPT = E // 32
CH = NSUP * SCH
EPTP = CH * KP
RPT = NPAD // 16


def _prep(X2, W1, W2, U, V):
    def body(x_ref, w1_ref, w2_ref, u_ref, v_ref, x_out, xv2t_out, xu_out):
        x = jnp.dot(x_ref[...], w1_ref[...], preferred_element_type=jnp.float32)
        x_out[...] = x
        xv2t_out[...] = jnp.dot(
            x, v_ref[...], preferred_element_type=jnp.float32) * w2_ref[...]
        xu_out[...] = jnp.dot(x, u_ref[...], preferred_element_type=jnp.float32)

    B = 2000
    return pl.pallas_call(
        body,
        grid=(N // B,),
        in_specs=[
            pl.BlockSpec((B, D), lambda i: (i, 0)),
            pl.BlockSpec((D, D), lambda i: (0, 0)),
            pl.BlockSpec((1, D), lambda i: (0, 0)),
            pl.BlockSpec((D, D), lambda i: (0, 0)),
            pl.BlockSpec((D, D), lambda i: (0, 0)),
        ],
        out_specs=[pl.BlockSpec((B, D), lambda i: (i, 0))] * 3,
        out_shape=[jax.ShapeDtypeStruct((N, D), jnp.float32)] * 3,
    )(X2, W1, W2, U, V)


def _sc_scatter(xv2t, src3, dst3, wp):
    mesh = plsc.VectorSubcoreMesh(core_axis_name="c", subcore_axis_name="s")

    @functools.partial(
        pl.kernel,
        out_type=jax.ShapeDtypeStruct((2, NPAD, AW), jnp.float32),
        mesh=mesh,
        compiler_params=pltpu.CompilerParams(
            needs_layout_passes=False, use_tc_tiling_on_sc=False),
        scratch_types=[
            pltpu.VMEM((2, SCH, KP), jnp.int32),
            pltpu.VMEM((2, SCH, KP), jnp.int32),
            pltpu.VMEM((2, SCH * KP), jnp.float32),
            pltpu.VMEM((2, KP, D), jnp.bfloat16),
            pltpu.VMEM((2, KP, AW), jnp.float32),
            pltpu.VMEM_SHARED((NPAD, AW), jnp.float32),
            pltpu.SemaphoreType.DMA,
            pltpu.SemaphoreType.DMA,
            pltpu.SemaphoreType.DMA,
            pltpu.SemaphoreType.DMA,
            pltpu.SemaphoreType.DMA,
            pltpu.SemaphoreType.DMA,
        ],
    )
    def k(xv2t_hbm, src_hbm, dst_hbm, w_hbm, out_hbm,
          src_sup, dst_sup, w_sup, gbuf, rows, accum,
          sg0, sg1, ss0, ss1, sm0, sm1):
        c = lax.axis_index("c")
        s = lax.axis_index("s")
        tid = c * 16 + s
        sem_g = (sg0, sg1)
        sem_s = (ss0, ss1)
        sem_m = (sm0, sm1)

        zero16 = jnp.zeros((16,), jnp.float32)

        @plsc.parallel_loop(0, KP, unroll=4)
        def _(e):
            for b in range(2):
                for j in range(8):
                    rows[b, e, pl.ds(j * 16, 16)] = zero16
                rows[b, e, pl.ds(AW - 16, 16)] = zero16

        for kk in range(RPT // KP):
            pltpu.sync_copy(rows.at[0], accum.at[pl.ds(s * RPT + kk * KP, KP)])

        tail = jnp.where(lax.iota(jnp.int32, 16) == 8,
                         jnp.float32(1.0), jnp.float32(0.0))

        @plsc.parallel_loop(0, KP, unroll=4)
        def _(e):
            rows[0, e, pl.ds(AW - 16, 16)] = tail
            rows[1, e, pl.ds(AW - 16, 16)] = tail

        plsc.subcore_barrier()

        def meta_start(ss, p):
            pltpu.async_copy(src_hbm.at[tid, ss], src_sup.at[p], sem_m[p])
            pltpu.async_copy(dst_hbm.at[tid, ss], dst_sup.at[p], sem_m[p])
            pltpu.async_copy(w_hbm.at[tid, ss], w_sup.at[p], sem_m[p])

        def meta_wait(ss, p):
            pltpu.make_async_copy(
                src_hbm.at[tid, ss], src_sup.at[p], sem_m[p]).wait()
            pltpu.make_async_copy(
                dst_hbm.at[tid, ss], dst_sup.at[p], sem_m[p]).wait()
            pltpu.make_async_copy(
                w_hbm.at[tid, ss], w_sup.at[p], sem_m[p]).wait()

        def gather_start(p, j, b):
            pltpu.async_copy(xv2t_hbm.at[src_sup.at[p, j]], gbuf.at[b],
                             sem_g[b])

        def gather_wait(b):
            pltpu.make_async_copy(
                xv2t_hbm.at[src_sup.at[0, 0]], gbuf.at[b], sem_g[b]).wait()

        def scatter_start(p, j, b):
            pltpu.async_copy(rows.at[b], accum.at[dst_sup.at[p, j]],
                             sem_s[b], add=True)

        def scatter_wait(b):
            pltpu.make_async_copy(
                rows.at[b], accum.at[pl.ds(0, KP)], sem_s[b]).wait()

        def scale(p, j, b):
            p16 = jnp.full((16,), p, jnp.int32)
            jbase = jnp.full((16,), j * KP, jnp.int32)

            @plsc.parallel_loop(0, KP, unroll=8)
            def _(e):
                wspl = plsc.load_gather(w_sup, [p16, jbase + e])
                for jj in range(4):
                    x32 = plsc.bitcast(gbuf[b, e, pl.ds(jj * 32, 32)],
                                       jnp.int32)
                    lo = plsc.bitcast(
                        lax.shift_left(x32, jnp.int32(16)), jnp.float32)
                    hi = plsc.bitcast(
                        lax.bitwise_and(x32, jnp.int32(-65536)), jnp.float32)
                    rows[b, e, pl.ds(jj * 32, 16)] = lo * wspl
                    rows[b, e, pl.ds(jj * 32 + 16, 16)] = hi * wspl

        meta_start(0, 0)
        meta_wait(0, 0)
        gather_start(0, 0, 0)
        gather_start(0, 1, 1)

        def suppair(g, carry):
            for sp in range(2):
                ss = 2 * g + sp
                for j in range(SCH):
                    b = j & 1
                    if j == 2:
                        meta_start(jnp.minimum(ss + 1, NSUP - 1), sp ^ 1)
                    if j >= 2:
                        scatter_wait(b)
                    else:
                        @pl.when(ss > 0)
                        def _():
                            scatter_wait(b)
                    gather_wait(b)
                    scale(sp, j, b)
                    if j == 6:
                        meta_wait(jnp.minimum(ss + 1, NSUP - 1), sp ^ 1)
                    if j < SCH - 2:
                        gather_start(sp, j + 2, b)
                    else:
                        gather_start(sp ^ 1, j + 2 - SCH, b)
                    scatter_start(sp, j, b)
            return carry

        lax.fori_loop(0, NSUP // 2, suppair, 0)

        scatter_wait(0)
        scatter_wait(1)
        gather_wait(0)
        gather_wait(1)
        plsc.subcore_barrier()
        pltpu.sync_copy(accum.at[pl.ds(s * RPT, RPT)],
                        out_hbm.at[c, pl.ds(s * RPT, RPT)])

    return k(xv2t, src3, dst3, wp)


def _post(x, xu, acc):
    def body(x_ref, xu_ref, a_ref, o_ref):
        a = a_ref[0] + a_ref[1]
        ssum = a[:, :D]
        cnt = a[:, D:D + 1]
        denom = jnp.maximum(cnt, 1.0)
        ao = xu_ref[...] + ssum / denom
        mu = jnp.mean(ao, axis=1, keepdims=True)
        var = jnp.mean(jnp.square(ao - mu), axis=1, keepdims=True)
        normed = (ao - mu) * lax.rsqrt(var + jnp.float32(1e-5))
        o_ref[...] = x_ref[...] + jnp.maximum(normed, 0.0)

    B = 2000
    return pl.pallas_call(
        body,
        grid=(N // B,),
        in_specs=[
            pl.BlockSpec((B, D), lambda i: (i, 0)),
            pl.BlockSpec((B, D), lambda i: (i, 0)),
            pl.BlockSpec((2, B, AW), lambda i: (0, i, 0)),
        ],
        out_specs=pl.BlockSpec((B, D), lambda i: (i, 0)),
        out_shape=jax.ShapeDtypeStruct((N, D), jnp.float32),
    )(x, xu, acc)


def kernel(X, edge_index, edge_weight, W1, W2, U, V):
    X2 = X.reshape(N, D)
    x, xv2t, xu = _prep(X2, W1, W2, U, V)

    pad = EPTP - EPT
    src3 = jnp.concatenate(
        [edge_index[0].reshape(32, EPT),
         jnp.zeros((32, pad), jnp.int32)], axis=1).reshape(32, NSUP, SCH, KP)
    dst3 = jnp.concatenate(
        [edge_index[1].reshape(32, EPT),
         jnp.full((32, pad), N, jnp.int32)], axis=1).reshape(32, NSUP, SCH, KP)
    wp = jnp.concatenate(
        [edge_weight.reshape(32, EPT),
         jnp.zeros((32, pad), jnp.float32)], axis=1).reshape(32, NSUP, SCH * KP)

    xv2b = (xv2t.reshape(N, 4, 2, 16).swapaxes(2, 3).reshape(N, D)
            .astype(jnp.bfloat16))
    acc = _sc_scatter(xv2b, src3, dst3, wp)
    out = _post(x, xu, acc)
    return out.reshape(1, N, D)
